# baseline (speedup 1.0000x reference)
import functools

import jax
import jax.numpy as jnp
from jax import lax
from jax.experimental import pallas as pl
from jax.experimental.pallas import tpu as pltpu
from jax.experimental.pallas import tpu_sc as plsc

N_HEADS = 16
FEAT = 8
N_ATOM = 10
N_DEG = 64
EMB = N_HEADS * FEAT
N = 100000

NW = 32
CHUNK = 3200
NP = NW * CHUNK
SUB = 640
IDX_ROWS = CHUNK // 128


def _combine_body(at_ref, dt_ref, c_ref):
    dt = dt_ref[...]
    dr = lax.broadcasted_iota(jnp.int32, (N_DEG, EMB), 0)
    dtm = jnp.where(dr == 0, 0.0, dt)
    c_ref[pl.ds(0, N_DEG), :] = dtm
    for a in range(1, N_ATOM):
        row = at_ref[pl.ds(a, 1), :]
        c_ref[pl.ds(a * N_DEG, N_DEG), :] = dtm + row


_build_combined = pl.pallas_call(
    _combine_body,
    out_shape=jax.ShapeDtypeStruct((N_ATOM * N_DEG, EMB), jnp.float32),
)


def _sc_body(atoms_hbm, degs_hbm, c_hbm, out_hbm, a_v, d_v, idx_v, rows_v,
             sem_g, sem_w):
    wid = lax.axis_index("s") * 2 + lax.axis_index("c")

    pltpu.sync_copy(atoms_hbm.at[wid], a_v)
    pltpu.sync_copy(degs_hbm.at[wid], d_v)

    def _idx_body(i, _):
        a = a_v[pl.ds(i * 16, 16)]
        d = d_v[pl.ds(i * 16, 16)]
        idx_v[i // 8, pl.ds((i % 8) * 16, 16)] = a * N_DEG + d
        return 0

    lax.fori_loop(0, CHUNK // 16, _idx_body, 0)

    def _do_chunk(r, n_idx_rows, n_write):
        cps = []
        for t in range(n_idx_rows):
            cps.append(pltpu.async_copy(
                c_hbm.at[idx_v.at[5 * r + t]],
                rows_v.at[pl.ds(t * 128, 128), :],
                sem_g))
        for cp in cps:
            cp.wait()
        base = wid * CHUNK + r * SUB
        wps = []
        for h in range(N_HEADS):
            wps.append(pltpu.async_copy(
                rows_v.at[pl.ds(0, n_write), pl.ds(h * FEAT, FEAT)],
                out_hbm.at[h, pl.ds(base, n_write), :],
                sem_w))
        for wp in wps:
            wp.wait()

    for r in range(CHUNK // SUB):
        base = wid * CHUNK + r * SUB
        full = base + SUB <= N

        @pl.when(full)
        def _():
            _do_chunk(r, SUB // 128, SUB)

        part = jnp.logical_and(jnp.logical_not(full), base < N)

        @pl.when(part)
        def _():
            _do_chunk(r, 2, N % SUB)


_sc_lookup = functools.partial(
    pl.kernel,
    out_type=jax.ShapeDtypeStruct((N_HEADS, N, FEAT), jnp.float32),
    mesh=plsc.VectorSubcoreMesh(core_axis_name="c", subcore_axis_name="s"),
    compiler_params=pltpu.CompilerParams(use_tc_tiling_on_sc=False),
    scratch_types=[
        pltpu.VMEM((CHUNK,), jnp.int32),
        pltpu.VMEM((CHUNK,), jnp.int32),
        pltpu.VMEM((IDX_ROWS, 128), jnp.int32),
        pltpu.VMEM((SUB, EMB), jnp.float32),
        pltpu.SemaphoreType.DMA,
        pltpu.SemaphoreType.DMA,
    ],
)(_sc_body)


def kernel(atoms, degrees, atom_table, degree_table):
    combined = _build_combined(atom_table, degree_table)
    pad = NP - N
    a2 = jnp.concatenate(
        [atoms, jnp.zeros((pad,), jnp.int32)]).reshape(NW, CHUNK)
    d2 = jnp.concatenate(
        [degrees, jnp.zeros((pad,), jnp.int32)]).reshape(NW, CHUNK)
    return _sc_lookup(a2, d2, combined)


# speedup vs baseline: 2.7534x; 2.7534x over previous
import functools

import jax
import jax.numpy as jnp
from jax import lax
from jax.experimental import pallas as pl
from jax.experimental.pallas import tpu as pltpu
from jax.experimental.pallas import tpu_sc as plsc

N_HEADS = 16
FEAT = 8
N_ATOM = 10
N_DEG = 64
EMB = N_HEADS * FEAT
N = 100000

NW = 32
CHUNK = 3328
HALF = CHUNK // 2
GROUPS = HALF // 16
BULK = (N // 128) * 128
LAST_BASE = BULK - CHUNK
TAIL = N - BULK
TGROUPS = TAIL // 16


def _combine_body(at_ref, dt_ref, c_ref):
    dt = dt_ref[...]
    dr = lax.broadcasted_iota(jnp.int32, (N_DEG, EMB), 0)
    dtm = jnp.where(dr == 0, 0.0, dt)
    c_ref[pl.ds(0, N_DEG), :] = dtm
    for a in range(1, N_ATOM):
        row = at_ref[pl.ds(a, 1), :]
        c_ref[pl.ds(a * N_DEG, N_DEG), :] = dtm + row


_build_combined = pl.pallas_call(
    _combine_body,
    out_shape=jax.ShapeDtypeStruct((N_ATOM * N_DEG, EMB), jnp.float32),
)


def _pidx(a, d):
    return a * (N_DEG * EMB) + d * EMB


def _sc_body(atoms_hbm, degs_hbm, ct_hbm, out_hbm, a_v, d_v, pidx_v, ct_v,
             stage0, stage1, tail_v, tstage, sem_t, sem_w0, sem_w1):
    wid = lax.axis_index("s") * 2 + lax.axis_index("c")
    base = pl.multiple_of(jnp.minimum(wid * CHUNK, LAST_BASE), 128)

    tcp = pltpu.async_copy(ct_hbm, ct_v, sem_t)
    pltpu.sync_copy(atoms_hbm.at[pl.ds(base, CHUNK)], a_v)
    pltpu.sync_copy(degs_hbm.at[pl.ds(base, CHUNK)], d_v)
    pltpu.sync_copy(atoms_hbm.at[pl.ds(BULK, TAIL)], tail_v.at[0])
    pltpu.sync_copy(degs_hbm.at[pl.ds(BULK, TAIL)], tail_v.at[1])

    def _idx_body(i, _):
        a = a_v[pl.ds(i * 16, 16)]
        d = d_v[pl.ds(i * 16, 16)]
        pidx_v[pl.ds(i * 16, 16)] = _pidx(a, d)
        return 0

    lax.fori_loop(0, CHUNK // 16, _idx_body, 0)
    for g in range(TGROUPS):
        tail_v[2, pl.ds(g * 16, 16)] = _pidx(tail_v[0, pl.ds(g * 16, 16)],
                                             tail_v[1, pl.ds(g * 16, 16)])
    tcp.wait()

    def _fill(stage, s, col0):
        def _g_body(g, _):
            p = pidx_v[pl.ds(s * HALF + g * 16, 16)]
            for f in range(FEAT):
                stage[f, pl.ds(g * 16, 16)] = plsc.load_gather(
                    ct_v, [p + (col0 + f)])
            return 0

        lax.fori_loop(0, GROUPS, _g_body, 0)

    stages = (stage0, stage1)
    sems = (sem_w0, sem_w1)
    pending = [None, None]
    for h in range(N_HEADS):
        for s in range(2):
            if pending[s] is not None:
                pending[s].wait()
            _fill(stages[s], s, h * FEAT)
            pending[s] = pltpu.async_copy(
                stages[s], out_hbm.at[h, :, pl.ds(base + s * HALF, HALF)],
                sems[s])
    for p in pending:
        p.wait()

    for h in range(N_HEADS):
        for g in range(TGROUPS):
            p = tail_v[2, pl.ds(g * 16, 16)]
            for f in range(FEAT):
                tstage[f, pl.ds(g * 16, 16)] = plsc.load_gather(
                    ct_v, [p + (h * FEAT + f)])
        pltpu.sync_copy(tstage, out_hbm.at[h, :, pl.ds(BULK, TAIL)])


_sc_lookup = functools.partial(
    pl.kernel,
    out_type=jax.ShapeDtypeStruct((N_HEADS, FEAT, N), jnp.float32),
    mesh=plsc.VectorSubcoreMesh(core_axis_name="c", subcore_axis_name="s"),
    compiler_params=pltpu.CompilerParams(use_tc_tiling_on_sc=True,
                                         needs_layout_passes=False),
    scratch_types=[
        pltpu.VMEM((CHUNK,), jnp.int32),
        pltpu.VMEM((CHUNK,), jnp.int32),
        pltpu.VMEM((CHUNK,), jnp.int32),
        pltpu.VMEM((N_ATOM * N_DEG * EMB,), jnp.float32),
        pltpu.VMEM((FEAT, HALF), jnp.float32),
        pltpu.VMEM((FEAT, HALF), jnp.float32),
        pltpu.VMEM((3, TAIL), jnp.int32),
        pltpu.VMEM((FEAT, TAIL), jnp.float32),
        pltpu.SemaphoreType.DMA,
        pltpu.SemaphoreType.DMA,
        pltpu.SemaphoreType.DMA,
    ],
)(_sc_body)


def kernel(atoms, degrees, atom_table, degree_table):
    combined = _build_combined(atom_table, degree_table)
    out = _sc_lookup(atoms, degrees, combined.reshape(-1))
    return jnp.transpose(out, (0, 2, 1))
